# baseline (speedup 1.0000x reference)
import jax
import jax.numpy as jnp
from jax import lax
from jax.experimental import pallas as pl
from jax.experimental.pallas import tpu as pltpu

TOPN = 64
N_NODES = 256


def _hgpool_body(a_ref, x_ref, eye_ref, an_ref, h_ref):
    A = a_ref[0]
    X = x_ref[0]
    eye = eye_ref[...]
    n = N_NODES
    hp = jax.lax.Precision.HIGHEST

    colsum = jnp.sum(A, axis=0, keepdims=True)
    s = (colsum ** (-1.0)).reshape(n, 1)

    D = s * eye
    W = eye - jnp.dot(D, A, preferred_element_type=jnp.float32)
    M = jnp.dot(W, X, preferred_element_type=jnp.float32)

    cur = jnp.abs(M)
    k = 1
    while k < n:
        cur = cur + jnp.roll(cur, -k, axis=1)
        k *= 2
    p_col = cur[:, 0:1]

    p_row = lax.dot_general(
        p_col, eye, (((0,), (0,)), ((), ())),
        precision=hp, preferred_element_type=jnp.float32,
    )

    ii = lax.broadcasted_iota(jnp.int32, (n, n), 0)
    jj = lax.broadcasted_iota(jnp.int32, (n, n), 1)
    before = (p_row < p_col) | ((p_row == p_col) & (jj < ii))
    rank = jnp.sum(before.astype(jnp.int32), axis=1)

    r_rows = lax.broadcasted_iota(jnp.int32, (TOPN, n), 0)
    S = (rank[None, :] == r_rows).astype(jnp.float32)
    r_cols = lax.broadcasted_iota(jnp.int32, (n, TOPN), 1)
    ST = (rank[:, None] == r_cols).astype(jnp.float32)

    H = jnp.dot(S, X, precision=hp, preferred_element_type=jnp.float32)
    SA = jnp.dot(S, A, precision=hp, preferred_element_type=jnp.float32)
    A_next = jnp.dot(SA, ST, precision=hp, preferred_element_type=jnp.float32)

    an_ref[0] = A_next
    h_ref[0] = H


def kernel(A, X):
    N, n = A.shape[0], A.shape[1]
    eye = jnp.eye(n, dtype=jnp.float32)
    out = pl.pallas_call(
        _hgpool_body,
        grid=(N,),
        in_specs=[
            pl.BlockSpec((1, n, n), lambda b: (b, 0, 0)),
            pl.BlockSpec((1, n, n), lambda b: (b, 0, 0)),
            pl.BlockSpec((n, n), lambda b: (0, 0)),
        ],
        out_specs=[
            pl.BlockSpec((1, TOPN, TOPN), lambda b: (b, 0, 0)),
            pl.BlockSpec((1, TOPN, n), lambda b: (b, 0, 0)),
        ],
        out_shape=[
            jax.ShapeDtypeStruct((N, TOPN, TOPN), jnp.float32),
            jax.ShapeDtypeStruct((N, TOPN, n), jnp.float32),
        ],
        compiler_params=pltpu.CompilerParams(
            dimension_semantics=("arbitrary",),
        ),
    )(A, X, eye)
    return (out[0], out[1])


# speedup vs baseline: 1.0408x; 1.0408x over previous
import jax
import jax.numpy as jnp
from jax import lax
from jax.experimental import pallas as pl
from jax.experimental.pallas import tpu as pltpu

TOPN = 64
N_NODES = 256


def _hgpool_body(a_ref, x_ref, eye_ref, an_ref, h_ref):
    A = a_ref[0]
    X = x_ref[0]
    eye = eye_ref[...]
    n = N_NODES
    hp = jax.lax.Precision.HIGHEST

    colsum = jnp.sum(A, axis=0, keepdims=True)
    s = (colsum ** (-1.0)).reshape(n, 1)

    s_b = s.astype(jnp.bfloat16).astype(jnp.float32)
    A_b = A.astype(jnp.bfloat16).astype(jnp.float32)
    W = eye - s_b * A_b
    M = jnp.dot(W, X, preferred_element_type=jnp.float32)

    cur = jnp.abs(M)
    k = 1
    while k < n:
        cur = cur + jnp.roll(cur, -k, axis=1)
        k *= 2
    p_col = cur[:, 0:1]

    p_row = lax.dot_general(
        p_col, eye, (((0,), (0,)), ((), ())),
        precision=hp, preferred_element_type=jnp.float32,
    )

    ii = lax.broadcasted_iota(jnp.int32, (n, n), 0)
    jj = lax.broadcasted_iota(jnp.int32, (n, n), 1)
    before = (p_row < p_col) | ((p_row == p_col) & (jj < ii))
    rank = jnp.sum(before.astype(jnp.int32), axis=1)

    r_rows = lax.broadcasted_iota(jnp.int32, (TOPN, n), 0)
    S = (rank[None, :] == r_rows).astype(jnp.float32)
    r_cols = lax.broadcasted_iota(jnp.int32, (n, TOPN), 1)
    ST = (rank[:, None] == r_cols).astype(jnp.float32)

    H = jnp.dot(S, X, precision=hp, preferred_element_type=jnp.float32)
    SA = jnp.dot(S, A, precision=hp, preferred_element_type=jnp.float32)
    A_next = jnp.dot(SA, ST, precision=hp, preferred_element_type=jnp.float32)

    an_ref[0] = A_next
    h_ref[0] = H


def kernel(A, X):
    N, n = A.shape[0], A.shape[1]
    eye = jnp.eye(n, dtype=jnp.float32)
    out = pl.pallas_call(
        _hgpool_body,
        grid=(N,),
        in_specs=[
            pl.BlockSpec((1, n, n), lambda b: (b, 0, 0)),
            pl.BlockSpec((1, n, n), lambda b: (b, 0, 0)),
            pl.BlockSpec((n, n), lambda b: (0, 0)),
        ],
        out_specs=[
            pl.BlockSpec((1, TOPN, TOPN), lambda b: (b, 0, 0)),
            pl.BlockSpec((1, TOPN, n), lambda b: (b, 0, 0)),
        ],
        out_shape=[
            jax.ShapeDtypeStruct((N, TOPN, TOPN), jnp.float32),
            jax.ShapeDtypeStruct((N, TOPN, n), jnp.float32),
        ],
        compiler_params=pltpu.CompilerParams(
            dimension_semantics=("arbitrary",),
        ),
    )(A, X, eye)
    return (out[0], out[1])
